# baseline (speedup 1.0000x reference)
import functools

import jax
import jax.numpy as jnp
from jax.experimental import pallas as pl
from jax.experimental.pallas import tpu as pltpu

Q = 1024
D = 32
W = 1024
PARTS = 4
PR = W // PARTS
TOPK = 80


def _nn_kernel(mp_ref, qsq_ref, lib_ref, ksq_ref,
               smap_ref, idx_ref, s_ref,
               run_val, run_blk, *, nsteps):
    step = pl.program_id(0)

    @pl.when(step == 0)
    def _init():
        run_val[:, :] = jnp.full((8, Q), 1e37, dtype=jnp.float32)
        run_blk[:, :] = jnp.zeros((8, Q), dtype=jnp.int32)

    rv = run_val[:, :]
    rb = run_blk[:, :]
    qsq = qsq_ref[:, :]
    vbase = step * (W // 8)

    for part in range(PARTS):
        base = part * PR
        r2 = jax.lax.dot_general(
            lib_ref[base:base + PR, :], mp_ref[:, :],
            (((1,), (0,)), ((), ())),
            preferred_element_type=jnp.float32)
        for v in range(PR // 8):
            sl = base + v * 8
            t = ksq_ref[sl:sl + 8, :] + qsq
            d2 = t + r2[v * 8:v * 8 + 8, :]
            gv = jnp.full((8, Q), vbase + sl // 8, dtype=jnp.int32)
            pred = d2 < rv
            rv = jnp.where(pred, d2, rv)
            rb = jnp.where(pred, gv, rb)

    run_val[:, :] = rv
    run_blk[:, :] = rb

    @pl.when(step == nsteps - 1)
    def _fini():
        rvf = run_val[:, :]
        m1 = jnp.min(rvf, axis=0, keepdims=True)
        srow = jax.lax.broadcasted_iota(jnp.int32, (8, Q), 0)
        gkey = run_blk[:, :] * 8 + srow
        cand = jnp.where(rvf == m1, gkey, jnp.int32(2**31 - 1))
        idx_ref[:, :] = jnp.min(cand, axis=0, keepdims=True)

        dist = jnp.sqrt(jnp.maximum(m1, 1e-12))
        smap_ref[:, :] = dist

        vb = jax.lax.bitcast_convert_type(dist, jnp.int32)
        lo = jnp.full((1, 1), -1, jnp.int32)
        hi = jnp.full((1, 1), 0x7f7fffff, jnp.int32)

        def bis(_, carry):
            lo, hi = carry
            mid = lo + jax.lax.shift_right_logical(hi - lo, 1)
            cnt = jnp.sum((vb > mid).astype(jnp.float32),
                          axis=(0, 1), keepdims=True)
            p = cnt < jnp.float32(TOPK)
            return jnp.where(p, lo, mid), jnp.where(p, mid, hi)

        lo, hi = jax.lax.fori_loop(0, 31, bis, (lo, hi))
        theta = jax.lax.bitcast_convert_type(hi, jnp.float32)
        gt = vb > hi
        cnt_gt = jnp.sum(gt.astype(jnp.float32), axis=(0, 1), keepdims=True)
        sum_gt = jnp.sum(jnp.where(gt, dist, 0.0), axis=(0, 1), keepdims=True)
        s_ref[:, :] = (sum_gt + (jnp.float32(TOPK) - cnt_gt) * theta) \
            / jnp.float32(TOPK)


def kernel(patch, patch_lib):
    k = patch_lib.shape[0]
    kp = pl.cdiv(k, W) * W
    nsteps = kp // W

    mp = patch.T * jnp.float32(-2.0)
    qsq = jnp.sum(patch * patch, axis=1)[None, :]
    lib = jnp.pad(patch_lib, ((0, kp - k), (0, 0)))
    ksq = jnp.pad(jnp.sum(patch_lib * patch_lib, axis=1)[:, None],
                  ((0, kp - k), (0, 0)), constant_values=1e30)

    smap_row, idx_row, s11 = pl.pallas_call(
        functools.partial(_nn_kernel, nsteps=nsteps),
        grid=(nsteps,),
        in_specs=[
            pl.BlockSpec((D, Q), lambda i: (0, 0)),
            pl.BlockSpec((1, Q), lambda i: (0, 0)),
            pl.BlockSpec((W, D), lambda i: (i, 0)),
            pl.BlockSpec((W, 1), lambda i: (i, 0)),
        ],
        out_specs=[
            pl.BlockSpec((1, Q), lambda i: (0, 0)),
            pl.BlockSpec((1, Q), lambda i: (0, 0)),
            pl.BlockSpec((1, 1), lambda i: (0, 0)),
        ],
        out_shape=[
            jax.ShapeDtypeStruct((1, Q), jnp.float32),
            jax.ShapeDtypeStruct((1, Q), jnp.int32),
            jax.ShapeDtypeStruct((1, 1), jnp.float32),
        ],
        scratch_shapes=[
            pltpu.VMEM((8, Q), jnp.float32),
            pltpu.VMEM((8, Q), jnp.int32),
        ],
    )(mp, qsq, lib, ksq)

    s_map = smap_row.reshape(1, 1, Q)
    min_idx = idx_row.reshape(Q)
    s = s11.reshape(())
    return (s_map, min_idx, s)


# speedup vs baseline: 1.2645x; 1.2645x over previous
import functools

import jax
import jax.numpy as jnp
from jax.experimental import pallas as pl
from jax.experimental.pallas import tpu as pltpu

Q = 1024
D = 32
W = 1024
PARTS = 4
PR = W // PARTS
TOPK = 80


def _fold_step(mp_ref, qsq_ref, lib_ref, ksq_ref, run_val, run_blk,
               vbase, nrows):
    qsq = qsq_ref[:, :]
    entries = []
    for part in range(PARTS):
        base = part * PR
        if base >= nrows * 8:
            break
        prows = min(PR, nrows * 8 - base)
        r2 = jax.lax.dot_general(
            lib_ref[base:base + prows, :], mp_ref[:, :],
            (((1,), (0,)), ((), ())),
            preferred_element_type=jnp.float32)
        for v in range(prows // 8):
            sl = base + v * 8
            t = ksq_ref[sl:sl + 8, :] + qsq
            d2 = t + r2[v * 8:v * 8 + 8, :]
            lid = jnp.full((8, Q), sl // 8, dtype=jnp.int32)
            entries.append((d2, lid))

    while len(entries) > 1:
        nxt = []
        for i in range(0, len(entries) - 1, 2):
            (av, ai), (bv, bi) = entries[i], entries[i + 1]
            pred = av <= bv
            nxt.append((jnp.where(pred, av, bv), jnp.where(pred, ai, bi)))
        if len(entries) % 2:
            nxt.append(entries[-1])
        entries = nxt
    wval, wloc = entries[0]

    rv = run_val[:, :]
    rb = run_blk[:, :]
    pred = wval < rv
    run_val[:, :] = jnp.where(pred, wval, rv)
    run_blk[:, :] = jnp.where(pred, wloc + vbase, rb)


def _nn_kernel(mp_ref, qsq_ref, lib_ref, ksq_ref,
               smap_ref, idx_ref, s_ref,
               run_val, run_blk, *, nsteps, tail_rows):
    step = pl.program_id(0)

    @pl.when(step == 0)
    def _init():
        run_val[:, :] = jnp.full((8, Q), 1e37, dtype=jnp.float32)
        run_blk[:, :] = jnp.zeros((8, Q), dtype=jnp.int32)

    vbase = step * (W // 8)

    @pl.when(step < nsteps - 1)
    def _full():
        _fold_step(mp_ref, qsq_ref, lib_ref, ksq_ref, run_val, run_blk,
                   vbase, W // 8)

    @pl.when(step == nsteps - 1)
    def _tail():
        _fold_step(mp_ref, qsq_ref, lib_ref, ksq_ref, run_val, run_blk,
                   vbase, tail_rows)

        rvf = run_val[:, :]
        m1 = jnp.min(rvf, axis=0, keepdims=True)
        srow = jax.lax.broadcasted_iota(jnp.int32, (8, Q), 0)
        gkey = run_blk[:, :] * 8 + srow
        cand = jnp.where(rvf == m1, gkey, jnp.int32(2**31 - 1))
        idx_ref[:, :] = jnp.min(cand, axis=0, keepdims=True)

        dist = jnp.sqrt(jnp.maximum(m1, 1e-12))
        smap_ref[:, :] = dist

        vb = jax.lax.bitcast_convert_type(dist, jnp.int32)
        lo = jnp.full((1, 1), -1, jnp.int32)
        hi = jnp.full((1, 1), 0x7f7fffff, jnp.int32)

        def bis(_, carry):
            lo, hi = carry
            mid = lo + jax.lax.shift_right_logical(hi - lo, 1)
            cnt = jnp.sum((vb > mid).astype(jnp.float32),
                          axis=(0, 1), keepdims=True)
            p = cnt < jnp.float32(TOPK)
            return jnp.where(p, lo, mid), jnp.where(p, mid, hi)

        lo, hi = jax.lax.fori_loop(0, 31, bis, (lo, hi))
        theta = jax.lax.bitcast_convert_type(hi, jnp.float32)
        gt = vb > hi
        cnt_gt = jnp.sum(gt.astype(jnp.float32), axis=(0, 1), keepdims=True)
        sum_gt = jnp.sum(jnp.where(gt, dist, 0.0), axis=(0, 1), keepdims=True)
        s_ref[:, :] = (sum_gt + (jnp.float32(TOPK) - cnt_gt) * theta) \
            / jnp.float32(TOPK)


def kernel(patch, patch_lib):
    k = patch_lib.shape[0]
    nsteps = pl.cdiv(k, W)
    tail_rows = (k - (nsteps - 1) * W) // 8

    mp = patch.T * jnp.float32(-2.0)
    qsq = jnp.sum(patch * patch, axis=1)[None, :]
    ksq = jnp.sum(patch_lib * patch_lib, axis=1)[:, None]

    smap_row, idx_row, s11 = pl.pallas_call(
        functools.partial(_nn_kernel, nsteps=nsteps, tail_rows=tail_rows),
        grid=(nsteps,),
        in_specs=[
            pl.BlockSpec((D, Q), lambda i: (0, 0)),
            pl.BlockSpec((1, Q), lambda i: (0, 0)),
            pl.BlockSpec((W, D), lambda i: (i, 0)),
            pl.BlockSpec((W, 1), lambda i: (i, 0)),
        ],
        out_specs=[
            pl.BlockSpec((1, Q), lambda i: (0, 0)),
            pl.BlockSpec((1, Q), lambda i: (0, 0)),
            pl.BlockSpec((1, 1), lambda i: (0, 0)),
        ],
        out_shape=[
            jax.ShapeDtypeStruct((1, Q), jnp.float32),
            jax.ShapeDtypeStruct((1, Q), jnp.int32),
            jax.ShapeDtypeStruct((1, 1), jnp.float32),
        ],
        scratch_shapes=[
            pltpu.VMEM((8, Q), jnp.float32),
            pltpu.VMEM((8, Q), jnp.int32),
        ],
    )(mp, qsq, patch_lib, ksq)

    s_map = smap_row.reshape(1, 1, Q)
    min_idx = idx_row.reshape(Q)
    s = s11.reshape(())
    return (s_map, min_idx, s)


# speedup vs baseline: 1.9628x; 1.5523x over previous
import functools

import jax
import jax.numpy as jnp
from jax.experimental import pallas as pl
from jax.experimental.pallas import tpu as pltpu

Q = 1024
D = 32
W = 2048
PARTS = 8
PR = W // PARTS
TOPK = 80


def _fold_step(mp_ref, qsq_ref, lib_ref, run_val, run_blk,
               vbase, nrows):
    qsq = qsq_ref[:, :]
    entries = []
    for part in range(PARTS):
        base = part * PR
        if base >= nrows * 8:
            break
        prows = min(PR, nrows * 8 - base)
        lib_p = lib_ref[base:base + prows, :]
        r2 = jax.lax.dot_general(
            lib_p, mp_ref[:, :],
            (((1,), (0,)), ((), ())),
            preferred_element_type=jnp.float32)
        ksq_p = jnp.sum(lib_p * lib_p, axis=1, keepdims=True)
        for v in range(prows // 8):
            sl = base + v * 8
            t = ksq_p[v * 8:v * 8 + 8, :] + qsq
            d2 = t + r2[v * 8:v * 8 + 8, :]
            lid = jnp.full((8, Q), sl // 8, dtype=jnp.int32)
            entries.append((d2, lid))

    while len(entries) > 1:
        nxt = []
        for i in range(0, len(entries) - 1, 2):
            (av, ai), (bv, bi) = entries[i], entries[i + 1]
            pred = av <= bv
            nxt.append((jnp.where(pred, av, bv), jnp.where(pred, ai, bi)))
        if len(entries) % 2:
            nxt.append(entries[-1])
        entries = nxt
    wval, wloc = entries[0]

    rv = run_val[:, :]
    rb = run_blk[:, :]
    pred = wval < rv
    run_val[:, :] = jnp.where(pred, wval, rv)
    run_blk[:, :] = jnp.where(pred, wloc + vbase, rb)


def _nn_kernel(mp_ref, qsq_ref, lib_ref,
               smap_ref, idx_ref, s_ref,
               run_val, run_blk, *, nsteps, tail_rows):
    step = pl.program_id(0)

    @pl.when(step == 0)
    def _init():
        run_val[:, :] = jnp.full((8, Q), 1e37, dtype=jnp.float32)
        run_blk[:, :] = jnp.zeros((8, Q), dtype=jnp.int32)

    vbase = step * (W // 8)

    @pl.when(step < nsteps - 1)
    def _full():
        _fold_step(mp_ref, qsq_ref, lib_ref, run_val, run_blk,
                   vbase, W // 8)

    @pl.when(step == nsteps - 1)
    def _tail():
        _fold_step(mp_ref, qsq_ref, lib_ref, run_val, run_blk,
                   vbase, tail_rows)

        rvf = run_val[:, :]
        m1 = jnp.min(rvf, axis=0, keepdims=True)
        srow = jax.lax.broadcasted_iota(jnp.int32, (8, Q), 0)
        gkey = run_blk[:, :] * 8 + srow
        cand = jnp.where(rvf == m1, gkey, jnp.int32(2**31 - 1))
        idx_ref[:, :] = jnp.min(cand, axis=0, keepdims=True)

        dist = jnp.sqrt(jnp.maximum(m1, 1e-12))
        smap_ref[:, :] = dist

        vb = jax.lax.bitcast_convert_type(dist, jnp.int32)
        lo = jnp.full((1, 1), -1, jnp.int32)
        hi = jnp.full((1, 1), 0x7f7fffff, jnp.int32)

        def bis(_, carry):
            lo, hi = carry
            mid = lo + jax.lax.shift_right_logical(hi - lo, 1)
            cnt = jnp.sum((vb > mid).astype(jnp.float32),
                          axis=(0, 1), keepdims=True)
            p = cnt < jnp.float32(TOPK)
            return jnp.where(p, lo, mid), jnp.where(p, mid, hi)

        lo, hi = jax.lax.fori_loop(0, 31, bis, (lo, hi))
        theta = jax.lax.bitcast_convert_type(hi, jnp.float32)
        gt = vb > hi
        cnt_gt = jnp.sum(gt.astype(jnp.float32), axis=(0, 1), keepdims=True)
        sum_gt = jnp.sum(jnp.where(gt, dist, 0.0), axis=(0, 1), keepdims=True)
        s_ref[:, :] = (sum_gt + (jnp.float32(TOPK) - cnt_gt) * theta) \
            / jnp.float32(TOPK)


def kernel(patch, patch_lib):
    k = patch_lib.shape[0]
    nsteps = pl.cdiv(k, W)
    tail_rows = (k - (nsteps - 1) * W) // 8

    mp = patch.T * jnp.float32(-2.0)
    qsq = jnp.sum(patch * patch, axis=1)[None, :]

    smap_row, idx_row, s11 = pl.pallas_call(
        functools.partial(_nn_kernel, nsteps=nsteps, tail_rows=tail_rows),
        grid=(nsteps,),
        in_specs=[
            pl.BlockSpec((D, Q), lambda i: (0, 0)),
            pl.BlockSpec((1, Q), lambda i: (0, 0)),
            pl.BlockSpec((W, D), lambda i: (i, 0)),
        ],
        out_specs=[
            pl.BlockSpec((1, Q), lambda i: (0, 0)),
            pl.BlockSpec((1, Q), lambda i: (0, 0)),
            pl.BlockSpec((1, 1), lambda i: (0, 0)),
        ],
        out_shape=[
            jax.ShapeDtypeStruct((1, Q), jnp.float32),
            jax.ShapeDtypeStruct((1, Q), jnp.int32),
            jax.ShapeDtypeStruct((1, 1), jnp.float32),
        ],
        scratch_shapes=[
            pltpu.VMEM((8, Q), jnp.float32),
            pltpu.VMEM((8, Q), jnp.int32),
        ],
    )(mp, qsq, patch_lib)

    s_map = smap_row.reshape(1, 1, Q)
    min_idx = idx_row.reshape(Q)
    s = s11.reshape(())
    return (s_map, min_idx, s)
